# baseline (speedup 1.0000x reference)
import functools

import jax
import jax.numpy as jnp
from jax import lax
from jax.experimental import pallas as pl
from jax.experimental.pallas import tpu as pltpu
from jax.experimental.pallas import tpu_sc as plsc

_VOCAB = 1000000
_EMBED = 32
_BATCH = 16384
_C = 5
_PACK = 4

_NC = 2
_NS = 16
_NW = _NC * _NS
_BW = _BATCH // _NW
_CW = _BW * _C
_CHUNK = 128
_NJ = _BW // _CHUNK


def _body(tgt_hbm, ctx_hbm, ttab_hbm, ctab_hbm, out_hbm,
          t_stage, c_stage, idx_t, q_t, idx_c, q_c,
          t_chunk, c_chunk, out_stage, sem):
  wid = lax.axis_index("s") * _NC + lax.axis_index("c")
  base = wid * _BW

  lanes = lax.iota(jnp.int32, 16)
  zeros16 = jnp.zeros((16,), jnp.int32)

  for s in range(_NJ):
    pltpu.sync_copy(tgt_hbm.at[pl.ds(base + s * _CHUNK, _CHUNK)], t_stage)
    pltpu.sync_copy(ctx_hbm.at[pl.ds(base + s * _CHUNK, _CHUNK)], c_stage)
    for i in range(_CHUNK // 16):
      b16 = i * 16 + lanes
      o = s * _CHUNK + i * 16
      tv = plsc.load_gather(t_stage, [b16, zeros16])
      idx_t[pl.ds(o, 16)] = lax.shift_right_logical(tv, 2)
      q_t[pl.ds(o, 16)] = lax.bitwise_and(tv, 3)
      for c in range(_C):
        cv = plsc.load_gather(c_stage, [b16, jnp.full((16,), c, jnp.int32)])
        idx_c[pl.ds(c * _BW + o, 16)] = lax.shift_right_logical(cv, 2)
        q_c[pl.ds(c * _BW + o, 16)] = lax.bitwise_and(cv, 3)

  for j in range(_NJ):
    pltpu.async_copy(
        ttab_hbm.at[idx_t.at[pl.ds(j * _CHUNK, _CHUNK)]], t_chunk, sem
    ).wait()

    for c in range(_C):
      pltpu.async_copy(
          ctab_hbm.at[idx_c.at[pl.ds(c * _BW + j * _CHUNK, _CHUNK)]],
          c_chunk, sem).wait()

      def grp(i, _, c=c, j=j):
        b16 = i * 16 + lanes
        tq32 = q_t[pl.ds(j * _CHUNK + i * 16, 16)] * _EMBED
        cq32 = q_c[pl.ds(c * _BW + j * _CHUNK + i * 16, 16)] * _EMBED
        acc = jnp.zeros((16,), jnp.float32)
        for e in range(_EMBED):
          w = plsc.load_gather(t_chunk, [b16, tq32 + e])
          x = plsc.load_gather(c_chunk, [b16, cq32 + e])
          acc = acc + w * x
        plsc.store_scatter(out_stage, [b16, jnp.full((16,), c, jnp.int32)],
                           acc)
        return ()

      lax.fori_loop(0, _CHUNK // 16, grp, ())

    pltpu.sync_copy(out_stage, out_hbm.at[pl.ds(base + j * _CHUNK, _CHUNK)])


@jax.jit
def _run(target, context, ttab2, ctab2):
  mesh = plsc.VectorSubcoreMesh(core_axis_name="c", subcore_axis_name="s")
  k = functools.partial(
      pl.kernel,
      mesh=mesh,
      compiler_params=pltpu.CompilerParams(needs_layout_passes=False),
      out_type=jax.ShapeDtypeStruct((_BATCH, _C), jnp.float32),
      scratch_types=[
          pltpu.VMEM((_CHUNK, 1), jnp.int32),
          pltpu.VMEM((_CHUNK, _C), jnp.int32),
          pltpu.VMEM((_BW,), jnp.int32),
          pltpu.VMEM((_BW,), jnp.int32),
          pltpu.VMEM((_CW,), jnp.int32),
          pltpu.VMEM((_CW,), jnp.int32),
          pltpu.VMEM((_CHUNK, 128), jnp.float32),
          pltpu.VMEM((_CHUNK, 128), jnp.float32),
          pltpu.VMEM((_CHUNK, _C), jnp.float32),
          pltpu.SemaphoreType.DMA,
      ],
  )(_body)
  return k(target, context, ttab2, ctab2)


def kernel(target, context, target_table, context_table):
  ttab2 = target_table.reshape(_VOCAB // _PACK, _EMBED * _PACK)
  ctab2 = context_table.reshape(_VOCAB // _PACK, _EMBED * _PACK)
  return _run(target, context, ttab2, ctab2)
